# baseline (speedup 1.0000x reference)
import functools

import jax
import jax.numpy as jnp
from jax import lax
from jax.experimental import pallas as pl
from jax.experimental.pallas import tpu as pltpu
from jax.experimental.pallas import tpu_sc as plsc

TOKEN_INDEX = 0
B, S, D = 4, 2048, 1024

_mesh = plsc.VectorSubcoreMesh(core_axis_name="c", subcore_axis_name="s")


@functools.partial(
    pl.kernel,
    mesh=_mesh,
    out_type=jax.ShapeDtypeStruct((B, D), jnp.float32),
)
def _extract(inp_hbm, out_hbm):
    cid = lax.axis_index("c")
    sid = lax.axis_index("s")
    wid = sid * 2 + cid

    @pl.when(wid < B)
    def _():
        pltpu.sync_copy(inp_hbm.at[wid, TOKEN_INDEX], out_hbm.at[wid])


def kernel(inputs):
    return _extract(inputs)


# speedup vs baseline: 1.0422x; 1.0422x over previous
import functools

import jax
import jax.numpy as jnp
from jax import lax
from jax.experimental import pallas as pl
from jax.experimental.pallas import tpu as pltpu
from jax.experimental.pallas import tpu_sc as plsc

TOKEN_INDEX = 0
B, S, D = 4, 2048, 1024

_mesh = plsc.ScalarSubcoreMesh(axis_name="c", num_cores=2)


@functools.partial(
    pl.kernel,
    mesh=_mesh,
    out_type=jax.ShapeDtypeStruct((B, D), jnp.float32),
)
def _extract(inp_hbm, out_hbm):
    cid = lax.axis_index("c")
    for i in range(2):
        row = cid * 2 + i
        pltpu.sync_copy(inp_hbm.at[row, TOKEN_INDEX], out_hbm.at[row])


def kernel(inputs):
    return _extract(inputs)


# speedup vs baseline: 14.4029x; 13.8202x over previous
import jax
import jax.numpy as jnp
from jax.experimental import pallas as pl
from jax.experimental.pallas import tpu as pltpu

TOKEN_INDEX = 0
B, S, D = 4, 2048, 1024


def _extract_body(in_hbm, out_ref, sem):
    copy = pltpu.make_async_copy(in_hbm.at[:, TOKEN_INDEX], out_ref, sem)
    copy.start()
    copy.wait()


def kernel(inputs):
    return pl.pallas_call(
        _extract_body,
        out_shape=jax.ShapeDtypeStruct((B, D), jnp.float32),
        in_specs=[pl.BlockSpec(memory_space=pl.ANY)],
        out_specs=pl.BlockSpec((B, D), lambda: (0, 0)),
        scratch_shapes=[pltpu.SemaphoreType.DMA],
    )(inputs)
